# baseline (speedup 1.0000x reference)
from math import sqrt

import jax
import jax.numpy as jnp
from jax import lax
from jax.experimental import pallas as pl
from jax.experimental.pallas import tpu as pltpu
from jax.experimental.pallas import tpu_sc as plsc

N = 35
E = N * N
P = 48
L = 16
NW = 16
EPW = 80
EP = NW * EPW
SEG = P * P // NW
X2OFF = P * P
REDW = X2OFF + P * L

_BN_SCALE = 1.0 / sqrt(1.0 + 0.001)

INTERPRET = False


def _sf(s):
    return lax.broadcast_in_dim(s, (L,), ())


def _rcp(c):
    r = 1.0 / c
    return r * (2.0 - c * r)


def _sig(z):
    return _rcp(1.0 + jnp.exp(-z))


def _sc_body(xf, w1, b1, root1, prm, soff, s16, doff, attr, out,
             xv, w1v, b1v, root1v, pv, sov, s16v, dov, atv,
             acc, rowv, segv, tv, invv, x1v, x2v, xbig, red):
    sid = lax.axis_index("s")
    base = sid * EPW
    pltpu.sync_copy(xf, xv)
    pltpu.sync_copy(w1, w1v)
    pltpu.sync_copy(b1, b1v)
    pltpu.sync_copy(root1, root1v)
    pltpu.sync_copy(prm, pv)
    pltpu.sync_copy(soff.at[pl.ds(base, EPW)], sov)
    pltpu.sync_copy(s16.at[pl.ds(base, EPW)], s16v)
    pltpu.sync_copy(doff.at[pl.ds(base, EPW)], dov)
    pltpu.sync_copy(attr.at[pl.ds(base, EPW)], atv)

    iota = lax.iota(jnp.int32, L)
    zero = jnp.zeros((L,), jnp.float32)
    cntvec = (iota == 3).astype(jnp.float32)
    onehot0 = (iota == 0).astype(jnp.float32)

    def zero_acc():
        for q in range(P * P // L):
            acc[pl.ds(q * L, L)] = zero

    def reduce_acc():
        pltpu.sync_copy(acc, xbig.at[sid])
        plsc.subcore_barrier()
        seg = sid * SEG
        for q in range(SEG // L):
            t = zero
            for k in range(NW):
                pltpu.sync_copy(xbig.at[k, pl.ds(seg + q * L, L)], tv)
                t = t + tv[...]
            segv[pl.ds(q * L, L)] = t
        pltpu.sync_copy(segv, red.at[pl.ds(seg, SEG)])
        plsc.subcore_barrier()

    zero_acc()

    def l1_edge(el, c):
        ev = _sf(el)
        sovec = plsc.load_gather(sov, [ev])
        at = plsc.load_gather(atv, [ev])
        m = [zero, zero, cntvec]
        for i in range(N):
            xsi = plsc.load_gather(xv, [sovec + i])
            for j in range(3):
                w = jnp.maximum(
                    at * w1v[i, pl.ds(j * L, L)] + b1v[i, pl.ds(j * L, L)],
                    0.0)
                m[j] = m[j] + xsi * w
        dvec = plsc.load_gather(dov, [ev]) + iota
        for j in range(3):
            plsc.addupdate_scatter(acc, [dvec + j * L], m[j])
        return c

    lax.fori_loop(0, EPW, l1_edge, 0)
    reduce_acc()

    for k in range(3):
        r = sid + NW * k
        pltpu.sync_copy(red.at[pl.ds(r * P, P)], rowv)
        cnt = plsc.load_gather(rowv, [_sf(N)])
        inv = _rcp(jnp.maximum(cnt, 1.0))
        invv[pl.ds(k * L, L)] = inv
        rvec = _sf(r * P)
        rt = [zero, zero, zero]
        for i in range(N):
            xri = plsc.load_gather(xv, [rvec + i])
            for j in range(3):
                rt[j] = rt[j] + xri * root1v[i, pl.ds(j * L, L)]
        for j in range(3):
            dsj = pl.ds(j * L, L)
            pre = rowv[dsj] * inv + rt[j] + pv[0, dsj]
            rowv[dsj] = _sig(pre * pv[1, dsj] + pv[2, dsj])
        pltpu.sync_copy(rowv, red.at[pl.ds(r * P, P)])
    plsc.subcore_barrier()
    pltpu.sync_copy(red.at[pl.ds(0, P * P)], x1v)

    zero_acc()

    def l2_edge(el, c):
        ev = _sf(el)
        sovec = plsc.load_gather(sov, [ev])
        at = plsc.load_gather(atv, [ev])
        ps = zero
        for j in range(3):
            dsj = pl.ds(j * L, L)
            w2 = jnp.maximum(at * pv[3, dsj] + pv[4, dsj], 0.0)
            x1r = plsc.load_gather(x1v, [sovec + (iota + j * L)])
            ps = ps + x1r * w2
        m2 = jnp.where(iota == 0, _sf(jnp.sum(ps)), 0.0)
        dvec = plsc.load_gather(dov, [ev]) + iota
        plsc.addupdate_scatter(acc, [dvec], m2)
        return c

    lax.fori_loop(0, EPW, l2_edge, 0)
    reduce_acc()

    for k in range(3):
        r = sid + NW * k
        pltpu.sync_copy(red.at[pl.ds(r * P, P)], rowv)
        s2 = plsc.load_gather(rowv, [_sf(0)])
        invvec = invv[pl.ds(k * L, L)]
        rs = zero
        for j in range(3):
            dsj = pl.ds(j * L, L)
            rs = rs + x1v[pl.ds(r * P + j * L, L)] * pv[5, dsj]
        pcv = pv[12, pl.ds(0, L)]
        pre2 = s2 * invvec + _sf(jnp.sum(rs)) + _sf(pcv[0])
        z2 = pre2 * _sf(pcv[1]) + _sf(pcv[2])
        tv[...] = _sig(z2)
        pltpu.sync_copy(tv, red.at[pl.ds(X2OFF + r * L, L)])
    plsc.subcore_barrier()
    pltpu.sync_copy(red.at[pl.ds(X2OFF, P * L)], x2v)

    zero_acc()

    def l3_edge(el, c):
        ev = _sf(el)
        at = plsc.load_gather(atv, [ev])
        s16vec = plsc.load_gather(s16v, [ev])
        x2s = plsc.load_gather(x2v, [s16vec])
        dvec = plsc.load_gather(dov, [ev]) + iota
        for j in range(3):
            dsj = pl.ds(j * L, L)
            w3 = jnp.maximum(at * pv[6, dsj] + pv[7, dsj], 0.0)
            plsc.addupdate_scatter(acc, [dvec + j * L], x2s * w3)
        return c

    lax.fori_loop(0, EPW, l3_edge, 0)
    reduce_acc()

    for k in range(3):
        r = sid + NW * k
        pltpu.sync_copy(red.at[pl.ds(r * P, P)], rowv)
        x2r = plsc.load_gather(x2v, [_sf(r * L)])
        invvec = invv[pl.ds(k * L, L)]
        for j in range(3):
            dsj = pl.ds(j * L, L)
            pre = rowv[dsj] * invvec + x2r * pv[8, dsj] + pv[9, dsj]
            x4 = _sig(pre * pv[10, dsj] + pv[11, dsj])
            rowv[dsj] = (x4 + x1v[pl.ds(r * P + j * L, L)]) * 0.5
        pltpu.sync_copy(rowv, red.at[pl.ds(r * P, P)])
    plsc.subcore_barrier()
    pltpu.sync_copy(red.at[pl.ds(0, P * P)], x1v)

    for k in range(3):
        r = sid + NW * k
        rvec = _sf(r)
        for j in range(3):
            colidx = (iota + j * L) * P + rvec
            col = plsc.load_gather(x1v, [colidx])
            rowv[pl.ds(j * L, L)] = \
                (x1v[pl.ds(r * P + j * L, L)] + col) * 0.5
        pltpu.sync_copy(rowv, out.at[r])


@jax.jit
def kernel(x, edge_index, edge_attr, c1_nnW, c1_nnb, c1_root, c1_bias,
           bn1_g, bn1_b, c2_nnW, c2_nnb, c2_root, c2_bias, bn2_g, bn2_b,
           c3_nnW, c3_nnb, c3_root, c3_bias, bn3_g, bn3_b):
    f32 = jnp.float32

    def padr(v, n=P):
        v = v.reshape(-1).astype(f32)
        return jnp.zeros((n,), f32).at[:v.shape[0]].set(v)

    def pad2(m):
        m = m.astype(f32)
        return jnp.zeros((N, P), f32).at[:, :m.shape[1]].set(m)

    src = edge_index[0].astype(jnp.int32)
    dst = edge_index[1].astype(jnp.int32)
    pad = EP - E
    srcp = jnp.concatenate([src, jnp.zeros((pad,), jnp.int32)])
    dstp = jnp.concatenate([dst, jnp.full((pad,), P - 1, jnp.int32)])
    attrp = jnp.concatenate([edge_attr.reshape(-1).astype(f32),
                             jnp.zeros((pad,), f32)])

    xf = pad2(x).reshape(-1)
    w1 = pad2(c1_nnW.reshape(N, N))
    b1 = pad2(c1_nnb.reshape(N, N))
    root1 = pad2(c1_root)
    prm = jnp.stack([
        padr(c1_bias), padr(bn1_g * _BN_SCALE), padr(bn1_b),
        padr(c2_nnW), padr(c2_nnb), padr(c2_root),
        padr(c3_nnW), padr(c3_nnb), padr(c3_root.reshape(-1)),
        padr(c3_bias), padr(bn3_g * _BN_SCALE), padr(bn3_b),
        padr(jnp.stack([c2_bias[0], bn2_g[0] * _BN_SCALE, bn2_b[0]])),
    ])

    mesh = plsc.VectorSubcoreMesh(core_axis_name="c", subcore_axis_name="s",
                                  num_cores=1, num_subcores=NW)
    run = pl.kernel(
        _sc_body,
        out_type=jax.ShapeDtypeStruct((P, P), f32),
        mesh=mesh,
        compiler_params=pltpu.CompilerParams(needs_layout_passes=False),
        scratch_types=[
            pltpu.VMEM((N * P,), f32),
            pltpu.VMEM((N, P), f32),
            pltpu.VMEM((N, P), f32),
            pltpu.VMEM((N, P), f32),
            pltpu.VMEM((13, P), f32),
            pltpu.VMEM((EPW,), jnp.int32),
            pltpu.VMEM((EPW,), jnp.int32),
            pltpu.VMEM((EPW,), jnp.int32),
            pltpu.VMEM((EPW,), f32),
            pltpu.VMEM((P * P,), f32),
            pltpu.VMEM((P,), f32),
            pltpu.VMEM((SEG,), f32),
            pltpu.VMEM((L,), f32),
            pltpu.VMEM((3 * L,), f32),
            pltpu.VMEM((P * P,), f32),
            pltpu.VMEM((P * L,), f32),
            pltpu.VMEM_SHARED((NW, P * P), f32),
            pltpu.VMEM_SHARED((REDW,), f32),
        ],
        interpret=INTERPRET,
    )
    out = run(xf, w1, b1, root1, prm,
              srcp * P, srcp * L, dstp * P, attrp)
    return out[:N, :N]


# speedup vs baseline: 1.5398x; 1.5398x over previous
from math import sqrt

import jax
import jax.numpy as jnp
from jax import lax
from jax.experimental import pallas as pl
from jax.experimental.pallas import tpu as pltpu
from jax.experimental.pallas import tpu_sc as plsc

N = 35
E = N * N
P = 48
L = 16
NW = 16
EPW = 80
EP = NW * EPW
ACCW = 4096
SEGP = ACCW // NW
X2OFF = P * P
REDW = ACCW

_BN_SCALE = 1.0 / sqrt(1.0 + 0.001)

INTERPRET = False


def _sf(s):
    return lax.broadcast_in_dim(s, (L,), ())


def _rcp(c):
    r = 1.0 / c
    return r * (2.0 - c * r)


def _sig(z):
    return _rcp(1.0 + jnp.exp(-z))


def _sc_body(xf, w1, b1, root1, prm, soff, s16, doff, attr, out,
             xv, w1v, b1v, root1v, pv, sov, s16v, dov, atv,
             acc, rowv, segv, tv, invv, x1v, x2v, xslab, xbig, red):
    sid = lax.axis_index("s")
    base = sid * EPW
    pltpu.sync_copy(xf, xv)
    pltpu.sync_copy(w1, w1v)
    pltpu.sync_copy(b1, b1v)
    pltpu.sync_copy(root1, root1v)
    pltpu.sync_copy(prm, pv)
    pltpu.sync_copy(soff.at[pl.ds(base, EPW)], sov)
    pltpu.sync_copy(s16.at[pl.ds(base, EPW)], s16v)
    pltpu.sync_copy(doff.at[pl.ds(base, EPW)], dov)
    pltpu.sync_copy(attr.at[pl.ds(base, EPW)], atv)

    iota = lax.iota(jnp.int32, L)
    zero = jnp.zeros((L,), jnp.float32)
    cntvec = (iota == 3).astype(jnp.float32)
    onehot0 = (iota == 0).astype(jnp.float32)

    def zero_acc():
        for q in range(ACCW // L):
            acc[pl.ds(q * L, L)] = zero

    def reduce_acc():
        pltpu.sync_copy(acc, xbig.at[sid])
        plsc.subcore_barrier()
        seg = pl.multiple_of(sid * SEGP, SEGP)
        pltpu.sync_copy(xbig.at[:, pl.ds(seg, SEGP)], xslab)
        for q in range(SEGP // L):
            t = zero
            for k in range(NW):
                t = t + xslab[k, pl.ds(q * L, L)]
            segv[pl.ds(q * L, L)] = t
        pltpu.sync_copy(segv, red.at[pl.ds(seg, SEGP)])
        plsc.subcore_barrier()

    zero_acc()

    def l1_edge(el, c):
        ev = _sf(el)
        sovec = plsc.load_gather(sov, [ev])
        at = plsc.load_gather(atv, [ev])
        m = [zero, zero, cntvec]
        for i in range(N):
            xsi = plsc.load_gather(xv, [sovec + i])
            for j in range(3):
                w = jnp.maximum(
                    at * w1v[i, pl.ds(j * L, L)] + b1v[i, pl.ds(j * L, L)],
                    0.0)
                m[j] = m[j] + xsi * w
        dvec = plsc.load_gather(dov, [ev]) + iota
        for j in range(3):
            plsc.addupdate_scatter(acc, [dvec + j * L], m[j])
        return c

    lax.fori_loop(0, EPW, l1_edge, 0)
    reduce_acc()

    for k in range(3):
        r = sid + NW * k
        pltpu.sync_copy(red.at[pl.ds(r * P, P)], rowv)
        cnt = plsc.load_gather(rowv, [_sf(N)])
        inv = _rcp(jnp.maximum(cnt, 1.0))
        invv[pl.ds(k * L, L)] = inv
        rvec = _sf(r * P)
        rt = [zero, zero, zero]
        for i in range(N):
            xri = plsc.load_gather(xv, [rvec + i])
            for j in range(3):
                rt[j] = rt[j] + xri * root1v[i, pl.ds(j * L, L)]
        for j in range(3):
            dsj = pl.ds(j * L, L)
            pre = rowv[dsj] * inv + rt[j] + pv[0, dsj]
            rowv[dsj] = _sig(pre * pv[1, dsj] + pv[2, dsj])
        pltpu.sync_copy(rowv, red.at[pl.ds(r * P, P)])
    plsc.subcore_barrier()
    pltpu.sync_copy(red.at[pl.ds(0, P * P)], x1v)

    zero_acc()

    def l2_edge(el, c):
        ev = _sf(el)
        sovec = plsc.load_gather(sov, [ev])
        at = plsc.load_gather(atv, [ev])
        ps = zero
        for j in range(3):
            dsj = pl.ds(j * L, L)
            w2 = jnp.maximum(at * pv[3, dsj] + pv[4, dsj], 0.0)
            x1r = plsc.load_gather(x1v, [sovec + (iota + j * L)])
            ps = ps + x1r * w2
        m2 = jnp.where(iota == 0, _sf(jnp.sum(ps)), 0.0)
        dvec = plsc.load_gather(dov, [ev]) + iota
        plsc.addupdate_scatter(acc, [dvec], m2)
        return c

    lax.fori_loop(0, EPW, l2_edge, 0)
    reduce_acc()

    for k in range(3):
        r = sid + NW * k
        pltpu.sync_copy(red.at[pl.ds(r * P, P)], rowv)
        s2 = plsc.load_gather(rowv, [_sf(0)])
        invvec = invv[pl.ds(k * L, L)]
        rs = zero
        for j in range(3):
            dsj = pl.ds(j * L, L)
            rs = rs + x1v[pl.ds(r * P + j * L, L)] * pv[5, dsj]
        pcv = pv[12, pl.ds(0, L)]
        pre2 = s2 * invvec + _sf(jnp.sum(rs)) + _sf(pcv[0])
        z2 = pre2 * _sf(pcv[1]) + _sf(pcv[2])
        tv[...] = _sig(z2)
        pltpu.sync_copy(tv, red.at[pl.ds(X2OFF + r * L, L)])
    plsc.subcore_barrier()
    pltpu.sync_copy(red.at[pl.ds(X2OFF, P * L)], x2v)

    zero_acc()

    def l3_edge(el, c):
        ev = _sf(el)
        at = plsc.load_gather(atv, [ev])
        s16vec = plsc.load_gather(s16v, [ev])
        x2s = plsc.load_gather(x2v, [s16vec])
        dvec = plsc.load_gather(dov, [ev]) + iota
        for j in range(3):
            dsj = pl.ds(j * L, L)
            w3 = jnp.maximum(at * pv[6, dsj] + pv[7, dsj], 0.0)
            plsc.addupdate_scatter(acc, [dvec + j * L], x2s * w3)
        return c

    lax.fori_loop(0, EPW, l3_edge, 0)
    reduce_acc()

    for k in range(3):
        r = sid + NW * k
        pltpu.sync_copy(red.at[pl.ds(r * P, P)], rowv)
        x2r = plsc.load_gather(x2v, [_sf(r * L)])
        invvec = invv[pl.ds(k * L, L)]
        for j in range(3):
            dsj = pl.ds(j * L, L)
            pre = rowv[dsj] * invvec + x2r * pv[8, dsj] + pv[9, dsj]
            x4 = _sig(pre * pv[10, dsj] + pv[11, dsj])
            rowv[dsj] = (x4 + x1v[pl.ds(r * P + j * L, L)]) * 0.5
        pltpu.sync_copy(rowv, red.at[pl.ds(r * P, P)])
    plsc.subcore_barrier()
    pltpu.sync_copy(red.at[pl.ds(0, P * P)], x1v)

    for k in range(3):
        r = sid + NW * k
        rvec = _sf(r)
        for j in range(3):
            colidx = (iota + j * L) * P + rvec
            col = plsc.load_gather(x1v, [colidx])
            rowv[pl.ds(j * L, L)] = \
                (x1v[pl.ds(r * P + j * L, L)] + col) * 0.5
        pltpu.sync_copy(rowv, out.at[r])


@jax.jit
def kernel(x, edge_index, edge_attr, c1_nnW, c1_nnb, c1_root, c1_bias,
           bn1_g, bn1_b, c2_nnW, c2_nnb, c2_root, c2_bias, bn2_g, bn2_b,
           c3_nnW, c3_nnb, c3_root, c3_bias, bn3_g, bn3_b):
    f32 = jnp.float32

    def padr(v, n=P):
        v = v.reshape(-1).astype(f32)
        return jnp.zeros((n,), f32).at[:v.shape[0]].set(v)

    def pad2(m):
        m = m.astype(f32)
        return jnp.zeros((N, P), f32).at[:, :m.shape[1]].set(m)

    src = edge_index[0].astype(jnp.int32)
    dst = edge_index[1].astype(jnp.int32)
    pad = EP - E
    srcp = jnp.concatenate([src, jnp.zeros((pad,), jnp.int32)])
    dstp = jnp.concatenate([dst, jnp.full((pad,), P - 1, jnp.int32)])
    attrp = jnp.concatenate([edge_attr.reshape(-1).astype(f32),
                             jnp.zeros((pad,), f32)])

    xf = pad2(x).reshape(-1)
    w1 = pad2(c1_nnW.reshape(N, N))
    b1 = pad2(c1_nnb.reshape(N, N))
    root1 = pad2(c1_root)
    prm = jnp.stack([
        padr(c1_bias), padr(bn1_g * _BN_SCALE), padr(bn1_b),
        padr(c2_nnW), padr(c2_nnb), padr(c2_root),
        padr(c3_nnW), padr(c3_nnb), padr(c3_root.reshape(-1)),
        padr(c3_bias), padr(bn3_g * _BN_SCALE), padr(bn3_b),
        padr(jnp.stack([c2_bias[0], bn2_g[0] * _BN_SCALE, bn2_b[0]])),
    ])

    mesh = plsc.VectorSubcoreMesh(core_axis_name="c", subcore_axis_name="s",
                                  num_cores=1, num_subcores=NW)
    run = pl.kernel(
        _sc_body,
        out_type=jax.ShapeDtypeStruct((P, P), f32),
        mesh=mesh,
        compiler_params=pltpu.CompilerParams(needs_layout_passes=False),
        scratch_types=[
            pltpu.VMEM((N * P,), f32),
            pltpu.VMEM((N, P), f32),
            pltpu.VMEM((N, P), f32),
            pltpu.VMEM((N, P), f32),
            pltpu.VMEM((13, P), f32),
            pltpu.VMEM((EPW,), jnp.int32),
            pltpu.VMEM((EPW,), jnp.int32),
            pltpu.VMEM((EPW,), jnp.int32),
            pltpu.VMEM((EPW,), f32),
            pltpu.VMEM((ACCW,), f32),
            pltpu.VMEM((P,), f32),
            pltpu.VMEM((SEGP,), f32),
            pltpu.VMEM((L,), f32),
            pltpu.VMEM((3 * L,), f32),
            pltpu.VMEM((P * P,), f32),
            pltpu.VMEM((P * L,), f32),
            pltpu.VMEM((NW, SEGP), f32),
            pltpu.VMEM_SHARED((NW, ACCW), f32),
            pltpu.VMEM_SHARED((REDW,), f32),
        ],
        interpret=INTERPRET,
    )
    out = run(xf, w1, b1, root1, prm,
              srcp * P, srcp * L, dstp * P, attrp)
    return out[:N, :N]
